# Initial kernel scaffold; baseline (speedup 1.0000x reference)
import jax
import jax.numpy as jnp
from jax.experimental import pallas as pl


def kernel(hidden_states, W, scale, per_expert_scale):
    raise NotImplementedError("write your pallas kernel here")


# speedup vs baseline: 1.5630x; 1.5630x over previous
import functools

import jax
import jax.numpy as jnp
from jax import lax
from jax.experimental import pallas as pl
from jax.experimental.pallas import tpu as pltpu
from jax.experimental.pallas import tpu_sc as plsc

H = 2048
E = 64
TOP_K = 2
T = 16384
EPS = 1e-06


_BT = 512


def _tc_logits_body(x_ref, w_ref, sc_ref, out_ref):
    x = x_ref[...]
    x2 = x * x
    acc = x2[:, 0:128]
    for k in range(1, H // 128):
        acc = acc + x2[:, 128 * k:128 * (k + 1)]
    s = acc
    for k in range(1, 16):
        s = s + pltpu.roll(acc, 128 - 8 * k, 1)
    t = s + pltpu.roll(s, 124, 1)
    t = t + pltpu.roll(t, 126, 1)
    t = t + pltpu.roll(t, 127, 1)
    mean = t[:, 0:1] * (1.0 / H)
    r = lax.rsqrt(mean + EPS)
    h = ((x * sc_ref[...]) * r) * (float(H) ** -0.5)
    out_ref[...] = lax.dot_general(
        w_ref[...].astype(jnp.bfloat16), h.astype(jnp.bfloat16),
        (((1,), (1,)), ((), ())),
        preferred_element_type=jnp.float32,
    )


def _tc_logits(x, w, sc2d):
    grid = (T // _BT,)
    return pl.pallas_call(
        _tc_logits_body,
        grid=grid,
        in_specs=[
            pl.BlockSpec((_BT, H), lambda i: (i, 0)),
            pl.BlockSpec((E, H), lambda i: (0, 0)),
            pl.BlockSpec((1, H), lambda i: (0, 0)),
        ],
        out_specs=pl.BlockSpec((E, _BT), lambda i: (0, i)),
        out_shape=jax.ShapeDtypeStruct((E, T), jnp.float32),
        compiler_params=pltpu.CompilerParams(
            dimension_semantics=("arbitrary",),
        ),
    )(x, w, sc2d)


_NC = 2
_NS = 16
_NW = _NC * _NS
_CHUNK = T // _NW
_GROUPS = _CHUNK // 16

@functools.cache
def _make_sc_router():
    mesh = plsc.VectorSubcoreMesh(
        core_axis_name="c", subcore_axis_name="s",
        num_cores=_NC, num_subcores=_NS,
    )
    return pl.kernel(
        _sc_router_body,
        out_type=(
            jax.ShapeDtypeStruct((T, TOP_K), jnp.float32),
            jax.ShapeDtypeStruct((T, TOP_K), jnp.int32),
        ),
        mesh=mesh,
        scratch_types=[
            pltpu.VMEM((E, _CHUNK), jnp.float32),
            pltpu.VMEM((E,), jnp.float32),
            pltpu.VMEM((_CHUNK, TOP_K), jnp.float32),
            pltpu.VMEM((_CHUNK, TOP_K), jnp.int32),
        ],
        compiler_params=pltpu.CompilerParams(
            needs_layout_passes=False, use_tc_tiling_on_sc=False,
        ),
    )


def _sc_router_body(lg_hbm, pes_hbm, out_w_hbm, out_i_hbm, lg_v, pes_v, w_v, i_v):
    wid = lax.axis_index("s") * _NC + lax.axis_index("c")
    base = wid * _CHUNK
    pltpu.sync_copy(lg_hbm.at[:, pl.ds(base, _CHUNK)], lg_v)
    pltpu.sync_copy(pes_hbm, pes_v)

    lanes = lax.iota(jnp.int32, 16)
    zeros16 = jnp.zeros((16,), jnp.int32)
    ones16 = jnp.ones((16,), jnp.int32)
    neg_inf = jnp.full((16,), -jnp.inf, jnp.float32)

    def group_body(g, carry):
        t0 = g * 16
        m1, m2 = neg_inf, neg_inf
        i1, i2 = zeros16, zeros16
        for e in range(E):
            v = lg_v[e, pl.ds(t0, 16)]
            e_vec = jnp.full((16,), e, jnp.int32)
            gt1 = v > m1
            gt2 = v > m2
            i2 = jnp.where(gt1, i1, jnp.where(gt2, e_vec, i2))
            m2 = jnp.where(gt1, m1, jnp.where(gt2, v, m2))
            i1 = jnp.where(gt1, e_vec, i1)
            m1 = jnp.where(gt1, v, m1)
        w1 = 1.0 / (1.0 + jnp.exp(m2 - m1))
        w2 = 1.0 - w1
        w1 = w1 * plsc.load_gather(pes_v, [i1])
        w2 = w2 * plsc.load_gather(pes_v, [i2])
        rows = t0 + lanes
        plsc.store_scatter(w_v, [rows, zeros16], w1)
        plsc.store_scatter(w_v, [rows, ones16], w2)
        plsc.store_scatter(i_v, [rows, zeros16], i1)
        plsc.store_scatter(i_v, [rows, ones16], i2)
        return carry

    lax.fori_loop(0, _GROUPS, group_body, 0)

    pltpu.sync_copy(w_v, out_w_hbm.at[pl.ds(base, _CHUNK), :])
    pltpu.sync_copy(i_v, out_i_hbm.at[pl.ds(base, _CHUNK), :])


def kernel(hidden_states, W, scale, per_expert_scale):
    logits_t = _tc_logits(hidden_states, W, scale.reshape(1, H))
    top_w, top_i = _make_sc_router()(logits_t, per_expert_scale)
    return (top_w, top_i)


# speedup vs baseline: 1.5651x; 1.0013x over previous
import functools

import jax
import jax.numpy as jnp
from jax import lax
from jax.experimental import pallas as pl
from jax.experimental.pallas import tpu as pltpu
from jax.experimental.pallas import tpu_sc as plsc

H = 2048
E = 64
TOP_K = 2
T = 16384
EPS = 1e-06


_BT = 512


def _tc_logits_body(x_ref, w_ref, sc_ref, out_ref):
    x = x_ref[...]
    x2 = x * x
    acc = x2[:, 0:128]
    for k in range(1, H // 128):
        acc = acc + x2[:, 128 * k:128 * (k + 1)]
    s = acc
    for k in range(1, 16):
        s = s + pltpu.roll(acc, 128 - 8 * k, 1)
    t = s + pltpu.roll(s, 124, 1)
    t = t + pltpu.roll(t, 126, 1)
    t = t + pltpu.roll(t, 127, 1)
    mean = t[:, 0:1] * (1.0 / H)
    r = lax.rsqrt(mean + EPS)
    h = ((x * r) * sc_ref[...]) * (float(H) ** -0.5)
    out_ref[...] = lax.dot_general(
        w_ref[...].astype(jnp.bfloat16), h.astype(jnp.bfloat16),
        (((1,), (1,)), ((), ())),
        preferred_element_type=jnp.float32,
    )


def _tc_logits(x, w, sc2d):
    grid = (T // _BT,)
    return pl.pallas_call(
        _tc_logits_body,
        grid=grid,
        in_specs=[
            pl.BlockSpec((_BT, H), lambda i: (i, 0)),
            pl.BlockSpec((E, H), lambda i: (0, 0)),
            pl.BlockSpec((1, H), lambda i: (0, 0)),
        ],
        out_specs=pl.BlockSpec((E, _BT), lambda i: (0, i)),
        out_shape=jax.ShapeDtypeStruct((E, T), jnp.float32),
        compiler_params=pltpu.CompilerParams(
            dimension_semantics=("arbitrary",),
        ),
    )(x, w, sc2d)


_NC = 2
_NS = 16
_NW = _NC * _NS
_CHUNK = T // _NW
_GROUPS = _CHUNK // 16

@functools.cache
def _make_sc_router():
    mesh = plsc.VectorSubcoreMesh(
        core_axis_name="c", subcore_axis_name="s",
        num_cores=_NC, num_subcores=_NS,
    )
    return pl.kernel(
        _sc_router_body,
        out_type=(
            jax.ShapeDtypeStruct((T, TOP_K), jnp.float32),
            jax.ShapeDtypeStruct((T, TOP_K), jnp.int32),
        ),
        mesh=mesh,
        scratch_types=[
            pltpu.VMEM((E, _CHUNK), jnp.float32),
            pltpu.VMEM((E,), jnp.float32),
            pltpu.VMEM((_CHUNK, TOP_K), jnp.float32),
            pltpu.VMEM((_CHUNK, TOP_K), jnp.int32),
        ],
        compiler_params=pltpu.CompilerParams(
            needs_layout_passes=False, use_tc_tiling_on_sc=False,
        ),
    )


def _sc_router_body(lg_hbm, pes_hbm, out_w_hbm, out_i_hbm, lg_v, pes_v, w_v, i_v):
    wid = lax.axis_index("s") * _NC + lax.axis_index("c")
    base = wid * _CHUNK
    pltpu.sync_copy(lg_hbm.at[:, pl.ds(base, _CHUNK)], lg_v)
    pltpu.sync_copy(pes_hbm, pes_v)

    lanes = lax.iota(jnp.int32, 16)
    zeros16 = jnp.zeros((16,), jnp.int32)
    ones16 = jnp.ones((16,), jnp.int32)
    neg_inf = jnp.full((16,), -jnp.inf, jnp.float32)

    def group_body(g, carry):
        t0 = g * 16
        m1, m2 = neg_inf, neg_inf
        i1, i2 = zeros16, zeros16
        for e in range(E):
            v = lg_v[e, pl.ds(t0, 16)]
            e_vec = jnp.full((16,), e, jnp.int32)
            gt1 = v > m1
            gt2 = v > m2
            i2 = jnp.where(gt1, i1, jnp.where(gt2, e_vec, i2))
            m2 = jnp.where(gt1, m1, jnp.where(gt2, v, m2))
            i1 = jnp.where(gt1, e_vec, i1)
            m1 = jnp.where(gt1, v, m1)
        w1 = 1.0 / (1.0 + jnp.exp(m2 - m1))
        w2 = 1.0 - w1
        w1 = w1 * plsc.load_gather(pes_v, [i1])
        w2 = w2 * plsc.load_gather(pes_v, [i2])
        rows = t0 + lanes
        plsc.store_scatter(w_v, [rows, zeros16], w1)
        plsc.store_scatter(w_v, [rows, ones16], w2)
        plsc.store_scatter(i_v, [rows, zeros16], i1)
        plsc.store_scatter(i_v, [rows, ones16], i2)
        return carry

    lax.fori_loop(0, _GROUPS, group_body, 0)

    pltpu.sync_copy(w_v, out_w_hbm.at[pl.ds(base, _CHUNK), :])
    pltpu.sync_copy(i_v, out_i_hbm.at[pl.ds(base, _CHUNK), :])


def kernel(hidden_states, W, scale, per_expert_scale):
    logits_t = _tc_logits(hidden_states, W, scale.reshape(1, H))
    top_w, top_i = _make_sc_router()(logits_t, per_expert_scale)
    return (top_w, top_i)
